# Initial kernel scaffold; baseline (speedup 1.0000x reference)
import jax
import jax.numpy as jnp
from jax.experimental import pallas as pl


def kernel(price_input, e, concept, Wih, Whh, bih, bhh, W1, b1, W2, b2, Wl, bl):
    raise NotImplementedError("write your pallas kernel here")


# speedup vs baseline: 101.8985x; 101.8985x over previous
import jax
import jax.numpy as jnp
from jax import lax
from jax.experimental import pallas as pl
from jax.experimental.pallas import tpu as pltpu
from jax.experimental.pallas import tpu_sc as plsc

N_NODES = 50000
N_INC = 800000
SEQ = 16
FIN = 6
H = 32
NCORE = 2
NSUB = 16
NTILE = NCORE * NSUB
PER_TILE = N_INC // NTILE
CHUNK = 125
NCHUNK = PER_TILE // CHUNK
IDX_BLK = 40
NREFILL = NCHUNK // IDX_BLK
IDX_ROWS = N_INC // CHUNK
N_PAD = 50048
ROWS_PER_TILE = N_PAD // NSUB
CW = 8
BLK = 2000
GRID = N_NODES // BLK

_SC_MESH = dict(core_axis_name="c", subcore_axis_name="s",
                num_cores=NCORE, num_subcores=NSUB)


def _sc_pass_body(table, idxg, idxs, zrows, out, acc, igv, isv, rows, sem):
    c = lax.axis_index("c")
    s = lax.axis_index("s")
    w = c * NSUB + s
    sl = pl.ds(s * ROWS_PER_TILE, ROWS_PER_TILE)
    pltpu.sync_copy(zrows, acc.at[sl])
    plsc.subcore_barrier()

    def outer(j, carry):
        base = w * NCHUNK + j * IDX_BLK
        pltpu.sync_copy(idxg.at[pl.ds(base, IDX_BLK)], igv)
        pltpu.sync_copy(idxs.at[pl.ds(base, IDX_BLK)], isv)

        def body(i, carry2):
            pltpu.async_copy(table.at[igv.at[i]], rows, sem).wait()
            pltpu.sync_copy(rows, acc.at[isv.at[i]], add=True)
            return carry2

        lax.fori_loop(0, IDX_BLK, body, 0)
        return carry

    lax.fori_loop(0, NREFILL, outer, 0)
    plsc.subcore_barrier()

    @pl.when(c == 0)
    def _():
        pltpu.sync_copy(acc.at[sl], out.at[0, sl])

    @pl.when(c == 1)
    def _():
        pltpu.sync_copy(acc.at[sl], out.at[1, sl])


def _sc_pass(table, idxg, idxs, zrows):
    return pl.kernel(
        _sc_pass_body,
        out_type=jax.ShapeDtypeStruct((NCORE, N_PAD, H), jnp.float32),
        mesh=plsc.VectorSubcoreMesh(**_SC_MESH),
        scratch_types=[
            pltpu.VMEM_SHARED((N_PAD, H), jnp.float32),
            pltpu.VMEM((IDX_BLK, CHUNK), jnp.int32),
            pltpu.VMEM((IDX_BLK, CHUNK), jnp.int32),
            pltpu.VMEM((CHUNK, H), jnp.float32),
            pltpu.SemaphoreType.DMA,
        ],
        compiler_params=pltpu.CompilerParams(use_tc_tiling_on_sc=False),
    )(table, idxg, idxs, zrows)


def _sc_counts_body(idxg, idxs, zrows, ones, outd, outb, accd, accb,
                    igv, isv, onev):
    c = lax.axis_index("c")
    s = lax.axis_index("s")
    w = c * NSUB + s
    sl = pl.ds(s * ROWS_PER_TILE, ROWS_PER_TILE)
    pltpu.sync_copy(zrows, accd.at[sl])
    pltpu.sync_copy(zrows, accb.at[sl])
    pltpu.sync_copy(ones, onev)
    plsc.subcore_barrier()

    def outer(j, carry):
        base = w * NCHUNK + j * IDX_BLK
        pltpu.sync_copy(idxg.at[pl.ds(base, IDX_BLK)], igv)
        pltpu.sync_copy(idxs.at[pl.ds(base, IDX_BLK)], isv)

        def body(i, carry2):
            pltpu.sync_copy(onev, accd.at[igv.at[i]], add=True)
            pltpu.sync_copy(onev, accb.at[isv.at[i]], add=True)
            return carry2

        lax.fori_loop(0, IDX_BLK, body, 0)
        return carry

    lax.fori_loop(0, NREFILL, outer, 0)
    plsc.subcore_barrier()

    @pl.when(c == 0)
    def _():
        pltpu.sync_copy(accd.at[sl], outd.at[0, sl])
        pltpu.sync_copy(accb.at[sl], outb.at[0, sl])

    @pl.when(c == 1)
    def _():
        pltpu.sync_copy(accd.at[sl], outd.at[1, sl])
        pltpu.sync_copy(accb.at[sl], outb.at[1, sl])


def _sc_counts(idxg, idxs):
    zrows = jnp.zeros((ROWS_PER_TILE, CW), jnp.float32)
    ones = jnp.ones((CHUNK, CW), jnp.float32)
    return pl.kernel(
        _sc_counts_body,
        out_type=(jax.ShapeDtypeStruct((NCORE, N_PAD, CW), jnp.float32),
                  jax.ShapeDtypeStruct((NCORE, N_PAD, CW), jnp.float32)),
        mesh=plsc.VectorSubcoreMesh(**_SC_MESH),
        scratch_types=[
            pltpu.VMEM_SHARED((N_PAD, CW), jnp.float32),
            pltpu.VMEM_SHARED((N_PAD, CW), jnp.float32),
            pltpu.VMEM((IDX_BLK, CHUNK), jnp.int32),
            pltpu.VMEM((IDX_BLK, CHUNK), jnp.int32),
            pltpu.VMEM((CHUNK, CW), jnp.float32),
        ],
        compiler_params=pltpu.CompilerParams(use_tc_tiling_on_sc=False),
    )(idxg, idxs, zrows, ones)


def _gru_body(x_ref, wih_ref, whh_ref, bih_ref, bhh_ref, w1_ref, out_ref):
    x = x_ref[...]
    wih = wih_ref[...]
    whh = whh_ref[...]
    bih = bih_ref[...]
    bhh = bhh_ref[...]
    h = jnp.zeros((BLK, H), jnp.float32)
    for t in range(SEQ):
        xt = x[:, t * FIN:(t + 1) * FIN]
        gi = jnp.dot(xt, wih, preferred_element_type=jnp.float32) + bih
        gh = jnp.dot(h, whh, preferred_element_type=jnp.float32) + bhh
        r = jax.nn.sigmoid(gi[:, :H] + gh[:, :H])
        z = jax.nn.sigmoid(gi[:, H:2 * H] + gh[:, H:2 * H])
        n = jnp.tanh(gi[:, 2 * H:] + r * gh[:, 2 * H:])
        h = (1.0 - z) * n + z * h
    o = jnp.where(h > 0, h, 0.01 * h)
    out_ref[...] = jnp.dot(o, w1_ref[...], preferred_element_type=jnp.float32)


def _tc_gru(x2, wih_t, whh_t, bih2, bhh2, w1m_t):
    return pl.pallas_call(
        _gru_body,
        grid=(GRID,),
        in_specs=[
            pl.BlockSpec((BLK, SEQ * FIN), lambda i: (i, 0)),
            pl.BlockSpec((FIN, 3 * H), lambda i: (0, 0)),
            pl.BlockSpec((H, 3 * H), lambda i: (0, 0)),
            pl.BlockSpec((1, 3 * H), lambda i: (0, 0)),
            pl.BlockSpec((1, 3 * H), lambda i: (0, 0)),
            pl.BlockSpec((H, H), lambda i: (0, 0)),
        ],
        out_specs=pl.BlockSpec((BLK, H), lambda i: (i, 0)),
        out_shape=jax.ShapeDtypeStruct((N_NODES, H), jnp.float32),
    )(x2, wih_t, whh_t, bih2, bhh2, w1m_t)


def _inv_body(cd_ref, cb_ref, dinv_ref, binv_ref):
    cd = cd_ref[0, :, :1] + cd_ref[1, :, :1]
    cb = cb_ref[0, :, :1] + cb_ref[1, :, :1]
    dinv_ref[...] = jnp.where(cd > 0, 1.0 / cd, 0.0)
    binv_ref[...] = jnp.where(cb > 0, 1.0 / cb, 0.0)


def _tc_inv(cd, cb):
    return pl.pallas_call(
        _inv_body,
        grid=(GRID,),
        in_specs=[
            pl.BlockSpec((NCORE, BLK, CW), lambda i: (0, i, 0)),
            pl.BlockSpec((NCORE, BLK, CW), lambda i: (0, i, 0)),
        ],
        out_specs=[
            pl.BlockSpec((BLK, 1), lambda i: (i, 0)),
            pl.BlockSpec((BLK, 1), lambda i: (i, 0)),
        ],
        out_shape=[jax.ShapeDtypeStruct((N_NODES, 1), jnp.float32),
                   jax.ShapeDtypeStruct((N_NODES, 1), jnp.float32)],
    )(cd, cb)


def _edge_flush_body(p_ref, binv_ref, out_ref):
    out_ref[...] = (p_ref[0] + p_ref[1]) * binv_ref[...]


def _tc_edge_flush(p, binv):
    return pl.pallas_call(
        _edge_flush_body,
        grid=(GRID,),
        in_specs=[
            pl.BlockSpec((NCORE, BLK, H), lambda i: (0, i, 0)),
            pl.BlockSpec((BLK, 1), lambda i: (i, 0)),
        ],
        out_specs=pl.BlockSpec((BLK, H), lambda i: (i, 0)),
        out_shape=jax.ShapeDtypeStruct((N_NODES, H), jnp.float32),
    )(p, binv)


def _node_flush_body(p_ref, dinv_ref, b_ref, w_ref, out_ref):
    y = (p_ref[0] + p_ref[1]) * dinv_ref[...] + b_ref[...]
    y = jnp.where(y > 0, y, 0.2 * y)
    out_ref[...] = jnp.dot(y, w_ref[...], preferred_element_type=jnp.float32)


def _tc_node_flush(p, dinv, b2d, w_t):
    return pl.pallas_call(
        _node_flush_body,
        grid=(GRID,),
        in_specs=[
            pl.BlockSpec((NCORE, BLK, H), lambda i: (0, i, 0)),
            pl.BlockSpec((BLK, 1), lambda i: (i, 0)),
            pl.BlockSpec((1, H), lambda i: (0, 0)),
            pl.BlockSpec((H, H), lambda i: (0, 0)),
        ],
        out_specs=pl.BlockSpec((BLK, H), lambda i: (i, 0)),
        out_shape=jax.ShapeDtypeStruct((N_NODES, H), jnp.float32),
    )(p, dinv, b2d, w_t)


def _final_body(p_ref, dinv_ref, b_ref, wl_ref, bl_ref, out_ref):
    y = (p_ref[0] + p_ref[1]) * dinv_ref[...] + b_ref[...]
    y = jnp.where(y > 0, y, 0.2 * y)
    o = jnp.dot(y, wl_ref[...], preferred_element_type=jnp.float32) + bl_ref[...]
    out_ref[...] = jnp.where(o > 0, o, 0.01 * o)


def _tc_final(p, dinv, b2d, wl_t, bl2d):
    n_out = wl_t.shape[1]
    return pl.pallas_call(
        _final_body,
        grid=(GRID,),
        in_specs=[
            pl.BlockSpec((NCORE, BLK, H), lambda i: (0, i, 0)),
            pl.BlockSpec((BLK, 1), lambda i: (i, 0)),
            pl.BlockSpec((1, H), lambda i: (0, 0)),
            pl.BlockSpec((H, n_out), lambda i: (0, 0)),
            pl.BlockSpec((1, n_out), lambda i: (0, 0)),
        ],
        out_specs=pl.BlockSpec((BLK, n_out), lambda i: (i, 0)),
        out_shape=jax.ShapeDtypeStruct((N_NODES, n_out), jnp.float32),
    )(p, dinv, b2d, wl_t, bl2d)


def kernel(price_input, e, concept, Wih, Whh, bih, bhh, W1, b1, W2, b2, Wl, bl):
    x2 = price_input.reshape(N_NODES, SEQ * FIN)
    w1m_t = W1.reshape(4, H, H).mean(axis=0).T
    t1 = _tc_gru(x2, Wih.T, Whh.T, bih.reshape(1, -1), bhh.reshape(1, -1), w1m_t)
    ig = e[0].reshape(IDX_ROWS, CHUNK)
    ie = e[1].reshape(IDX_ROWS, CHUNK)
    cd, cb = _sc_counts(ig, ie)
    dinv, binv = _tc_inv(cd, cb)
    z = jnp.zeros((ROWS_PER_TILE, H), jnp.float32)
    p = _sc_pass(t1, ig, ie, z)
    t2 = _tc_edge_flush(p, binv)
    p = _sc_pass(t2, ie, ig, z)
    t3 = _tc_node_flush(p, dinv, b1.reshape(1, -1), W2.T)
    p = _sc_pass(t3, ig, ie, z)
    t4 = _tc_edge_flush(p, binv)
    p = _sc_pass(t4, ie, ig, z)
    return _tc_final(p, dinv, b2.reshape(1, -1), Wl.T, bl.reshape(1, -1))


# speedup vs baseline: 125.4233x; 1.2309x over previous
import jax
import jax.numpy as jnp
from jax import lax
from jax.experimental import pallas as pl
from jax.experimental.pallas import tpu as pltpu
from jax.experimental.pallas import tpu_sc as plsc

N_NODES = 50000
N_INC = 800000
SEQ = 16
FIN = 6
H = 32
NCORE = 2
NSUB = 16
NTILE = NCORE * NSUB
PER_TILE = N_INC // NTILE
CHUNK = 125
NCHUNK = PER_TILE // CHUNK
IDX_BLK = 40
NREFILL = NCHUNK // IDX_BLK
IDX_ROWS = N_INC // CHUNK
N_PAD = 50048
ROWS_PER_TILE = N_PAD // NSUB
CW = 8
BLK = 2000
GRID = N_NODES // BLK

_SC_MESH = dict(core_axis_name="c", subcore_axis_name="s",
                num_cores=NCORE, num_subcores=NSUB)


def _sc_pass_body(table, idxg, idxs, zrows, out, acc,
                  igv, isv, rows0, rows1, sem0, sem1):
    c = lax.axis_index("c")
    s = lax.axis_index("s")
    w = c * NSUB + s
    sl = pl.ds(s * ROWS_PER_TILE, ROWS_PER_TILE)
    pltpu.sync_copy(zrows, acc.at[sl])
    plsc.subcore_barrier()

    def outer(j, carry):
        base = w * NCHUNK + j * IDX_BLK
        pltpu.sync_copy(idxg.at[pl.ds(base, IDX_BLK)], igv)
        pltpu.sync_copy(idxs.at[pl.ds(base, IDX_BLK)], isv)
        pltpu.async_copy(table.at[igv.at[0]], rows0, sem0)

        def body(k, carry2):
            i0 = 2 * k
            pltpu.async_copy(table.at[igv.at[i0 + 1]], rows1, sem1)
            pltpu.make_async_copy(table.at[igv.at[i0]], rows0, sem0).wait()
            pltpu.sync_copy(rows0, acc.at[isv.at[i0]], add=True)

            @pl.when(k < IDX_BLK // 2 - 1)
            def _():
                pltpu.async_copy(table.at[igv.at[i0 + 2]], rows0, sem0)

            pltpu.make_async_copy(table.at[igv.at[i0 + 1]], rows1, sem1).wait()
            pltpu.sync_copy(rows1, acc.at[isv.at[i0 + 1]], add=True)
            return carry2

        lax.fori_loop(0, IDX_BLK // 2, body, 0)
        return carry

    lax.fori_loop(0, NREFILL, outer, 0)
    plsc.subcore_barrier()

    @pl.when(c == 0)
    def _():
        pltpu.sync_copy(acc.at[sl], out.at[0, sl])

    @pl.when(c == 1)
    def _():
        pltpu.sync_copy(acc.at[sl], out.at[1, sl])


def _sc_pass(table, idxg, idxs, zrows):
    return pl.kernel(
        _sc_pass_body,
        out_type=jax.ShapeDtypeStruct((NCORE, N_PAD, H), jnp.float32),
        mesh=plsc.VectorSubcoreMesh(**_SC_MESH),
        scratch_types=[
            pltpu.VMEM_SHARED((N_PAD, H), jnp.float32),
            pltpu.VMEM((IDX_BLK, CHUNK), jnp.int32),
            pltpu.VMEM((IDX_BLK, CHUNK), jnp.int32),
            pltpu.VMEM((CHUNK, H), jnp.float32),
            pltpu.VMEM((CHUNK, H), jnp.float32),
            pltpu.SemaphoreType.DMA,
            pltpu.SemaphoreType.DMA,
        ],
        compiler_params=pltpu.CompilerParams(use_tc_tiling_on_sc=False),
    )(table, idxg, idxs, zrows)


def _sc_counts_body(idxg, idxs, zrows, ones, outd, outb, accd, accb,
                    igv, isv, onev):
    c = lax.axis_index("c")
    s = lax.axis_index("s")
    w = c * NSUB + s
    sl = pl.ds(s * ROWS_PER_TILE, ROWS_PER_TILE)
    pltpu.sync_copy(zrows, accd.at[sl])
    pltpu.sync_copy(zrows, accb.at[sl])
    pltpu.sync_copy(ones, onev)
    plsc.subcore_barrier()

    def outer(j, carry):
        base = w * NCHUNK + j * IDX_BLK
        pltpu.sync_copy(idxg.at[pl.ds(base, IDX_BLK)], igv)
        pltpu.sync_copy(idxs.at[pl.ds(base, IDX_BLK)], isv)

        def body(i, carry2):
            pltpu.sync_copy(onev, accd.at[igv.at[i]], add=True)
            pltpu.sync_copy(onev, accb.at[isv.at[i]], add=True)
            return carry2

        lax.fori_loop(0, IDX_BLK, body, 0)
        return carry

    lax.fori_loop(0, NREFILL, outer, 0)
    plsc.subcore_barrier()

    @pl.when(c == 0)
    def _():
        pltpu.sync_copy(accd.at[sl], outd.at[0, sl])
        pltpu.sync_copy(accb.at[sl], outb.at[0, sl])

    @pl.when(c == 1)
    def _():
        pltpu.sync_copy(accd.at[sl], outd.at[1, sl])
        pltpu.sync_copy(accb.at[sl], outb.at[1, sl])


def _sc_counts(idxg, idxs):
    zrows = jnp.zeros((ROWS_PER_TILE, CW), jnp.float32)
    ones = jnp.ones((CHUNK, CW), jnp.float32)
    return pl.kernel(
        _sc_counts_body,
        out_type=(jax.ShapeDtypeStruct((NCORE, N_PAD, CW), jnp.float32),
                  jax.ShapeDtypeStruct((NCORE, N_PAD, CW), jnp.float32)),
        mesh=plsc.VectorSubcoreMesh(**_SC_MESH),
        scratch_types=[
            pltpu.VMEM_SHARED((N_PAD, CW), jnp.float32),
            pltpu.VMEM_SHARED((N_PAD, CW), jnp.float32),
            pltpu.VMEM((IDX_BLK, CHUNK), jnp.int32),
            pltpu.VMEM((IDX_BLK, CHUNK), jnp.int32),
            pltpu.VMEM((CHUNK, CW), jnp.float32),
        ],
        compiler_params=pltpu.CompilerParams(use_tc_tiling_on_sc=False),
    )(idxg, idxs, zrows, ones)


def _gru_body(x_ref, wih_ref, whh_ref, bih_ref, bhh_ref, w1_ref, out_ref):
    x = x_ref[...]
    wih = wih_ref[...]
    whh = whh_ref[...]
    bih = bih_ref[...]
    bhh = bhh_ref[...]
    h = jnp.zeros((BLK, H), jnp.float32)
    for t in range(SEQ):
        xt = x[:, t * FIN:(t + 1) * FIN]
        gi = jnp.dot(xt, wih, preferred_element_type=jnp.float32) + bih
        gh = jnp.dot(h, whh, preferred_element_type=jnp.float32) + bhh
        r = jax.nn.sigmoid(gi[:, :H] + gh[:, :H])
        z = jax.nn.sigmoid(gi[:, H:2 * H] + gh[:, H:2 * H])
        n = jnp.tanh(gi[:, 2 * H:] + r * gh[:, 2 * H:])
        h = (1.0 - z) * n + z * h
    o = jnp.where(h > 0, h, 0.01 * h)
    out_ref[...] = jnp.dot(o, w1_ref[...], preferred_element_type=jnp.float32)


def _tc_gru(x2, wih_t, whh_t, bih2, bhh2, w1m_t):
    return pl.pallas_call(
        _gru_body,
        grid=(GRID,),
        in_specs=[
            pl.BlockSpec((BLK, SEQ * FIN), lambda i: (i, 0)),
            pl.BlockSpec((FIN, 3 * H), lambda i: (0, 0)),
            pl.BlockSpec((H, 3 * H), lambda i: (0, 0)),
            pl.BlockSpec((1, 3 * H), lambda i: (0, 0)),
            pl.BlockSpec((1, 3 * H), lambda i: (0, 0)),
            pl.BlockSpec((H, H), lambda i: (0, 0)),
        ],
        out_specs=pl.BlockSpec((BLK, H), lambda i: (i, 0)),
        out_shape=jax.ShapeDtypeStruct((N_NODES, H), jnp.float32),
    )(x2, wih_t, whh_t, bih2, bhh2, w1m_t)


def _inv_body(cd_ref, cb_ref, dinv_ref, binv_ref):
    cd = cd_ref[0, :, :1] + cd_ref[1, :, :1]
    cb = cb_ref[0, :, :1] + cb_ref[1, :, :1]
    dinv_ref[...] = jnp.where(cd > 0, 1.0 / cd, 0.0)
    binv_ref[...] = jnp.where(cb > 0, 1.0 / cb, 0.0)


def _tc_inv(cd, cb):
    return pl.pallas_call(
        _inv_body,
        grid=(GRID,),
        in_specs=[
            pl.BlockSpec((NCORE, BLK, CW), lambda i: (0, i, 0)),
            pl.BlockSpec((NCORE, BLK, CW), lambda i: (0, i, 0)),
        ],
        out_specs=[
            pl.BlockSpec((BLK, 1), lambda i: (i, 0)),
            pl.BlockSpec((BLK, 1), lambda i: (i, 0)),
        ],
        out_shape=[jax.ShapeDtypeStruct((N_NODES, 1), jnp.float32),
                   jax.ShapeDtypeStruct((N_NODES, 1), jnp.float32)],
    )(cd, cb)


def _edge_flush_body(p_ref, binv_ref, out_ref):
    out_ref[...] = (p_ref[0] + p_ref[1]) * binv_ref[...]


def _tc_edge_flush(p, binv):
    return pl.pallas_call(
        _edge_flush_body,
        grid=(GRID,),
        in_specs=[
            pl.BlockSpec((NCORE, BLK, H), lambda i: (0, i, 0)),
            pl.BlockSpec((BLK, 1), lambda i: (i, 0)),
        ],
        out_specs=pl.BlockSpec((BLK, H), lambda i: (i, 0)),
        out_shape=jax.ShapeDtypeStruct((N_NODES, H), jnp.float32),
    )(p, binv)


def _node_flush_body(p_ref, dinv_ref, b_ref, w_ref, out_ref):
    y = (p_ref[0] + p_ref[1]) * dinv_ref[...] + b_ref[...]
    y = jnp.where(y > 0, y, 0.2 * y)
    out_ref[...] = jnp.dot(y, w_ref[...], preferred_element_type=jnp.float32)


def _tc_node_flush(p, dinv, b2d, w_t):
    return pl.pallas_call(
        _node_flush_body,
        grid=(GRID,),
        in_specs=[
            pl.BlockSpec((NCORE, BLK, H), lambda i: (0, i, 0)),
            pl.BlockSpec((BLK, 1), lambda i: (i, 0)),
            pl.BlockSpec((1, H), lambda i: (0, 0)),
            pl.BlockSpec((H, H), lambda i: (0, 0)),
        ],
        out_specs=pl.BlockSpec((BLK, H), lambda i: (i, 0)),
        out_shape=jax.ShapeDtypeStruct((N_NODES, H), jnp.float32),
    )(p, dinv, b2d, w_t)


def _final_body(p_ref, dinv_ref, b_ref, wl_ref, bl_ref, out_ref):
    y = (p_ref[0] + p_ref[1]) * dinv_ref[...] + b_ref[...]
    y = jnp.where(y > 0, y, 0.2 * y)
    o = jnp.dot(y, wl_ref[...], preferred_element_type=jnp.float32) + bl_ref[...]
    out_ref[...] = jnp.where(o > 0, o, 0.01 * o)


def _tc_final(p, dinv, b2d, wl_t, bl2d):
    n_out = wl_t.shape[1]
    return pl.pallas_call(
        _final_body,
        grid=(GRID,),
        in_specs=[
            pl.BlockSpec((NCORE, BLK, H), lambda i: (0, i, 0)),
            pl.BlockSpec((BLK, 1), lambda i: (i, 0)),
            pl.BlockSpec((1, H), lambda i: (0, 0)),
            pl.BlockSpec((H, n_out), lambda i: (0, 0)),
            pl.BlockSpec((1, n_out), lambda i: (0, 0)),
        ],
        out_specs=pl.BlockSpec((BLK, n_out), lambda i: (i, 0)),
        out_shape=jax.ShapeDtypeStruct((N_NODES, n_out), jnp.float32),
    )(p, dinv, b2d, wl_t, bl2d)


def kernel(price_input, e, concept, Wih, Whh, bih, bhh, W1, b1, W2, b2, Wl, bl):
    x2 = price_input.reshape(N_NODES, SEQ * FIN)
    w1m_t = W1.reshape(4, H, H).mean(axis=0).T
    t1 = _tc_gru(x2, Wih.T, Whh.T, bih.reshape(1, -1), bhh.reshape(1, -1), w1m_t)
    ig = e[0].reshape(IDX_ROWS, CHUNK)
    ie = e[1].reshape(IDX_ROWS, CHUNK)
    cd, cb = _sc_counts(ig, ie)
    dinv, binv = _tc_inv(cd, cb)
    z = jnp.zeros((ROWS_PER_TILE, H), jnp.float32)
    p = _sc_pass(t1, ig, ie, z)
    t2 = _tc_edge_flush(p, binv)
    p = _sc_pass(t2, ie, ig, z)
    t3 = _tc_node_flush(p, dinv, b1.reshape(1, -1), W2.T)
    p = _sc_pass(t3, ig, ie, z)
    t4 = _tc_edge_flush(p, binv)
    p = _sc_pass(t4, ie, ig, z)
    return _tc_final(p, dinv, b2.reshape(1, -1), Wl.T, bl.reshape(1, -1))


# speedup vs baseline: 132.2610x; 1.0545x over previous
import jax
import jax.numpy as jnp
from jax import lax
from jax.experimental import pallas as pl
from jax.experimental.pallas import tpu as pltpu
from jax.experimental.pallas import tpu_sc as plsc

N_NODES = 50000
N_INC = 800000
SEQ = 16
FIN = 6
H = 32
NCORE = 2
NSUB = 16
NTILE = NCORE * NSUB
PER_TILE = N_INC // NTILE
CHUNK = 125
NCHUNK = PER_TILE // CHUNK
IDX_BLK = 40
NREFILL = NCHUNK // IDX_BLK
IDX_ROWS = N_INC // CHUNK
N_PAD = 50048
ROWS_PER_TILE = N_PAD // NSUB
CW = 8
BLK = 2000
GRID = N_NODES // BLK

_SC_MESH = dict(core_axis_name="c", subcore_axis_name="s",
                num_cores=NCORE, num_subcores=NSUB)


def _sc_pass_body(table, idxg, idxs, zrows, out, acc,
                  igv, isv, rows0, rows1, sem0, sem1):
    c = lax.axis_index("c")
    s = lax.axis_index("s")
    w = c * NSUB + s
    sl = pl.ds(s * ROWS_PER_TILE, ROWS_PER_TILE)
    pltpu.sync_copy(zrows, acc.at[sl])
    plsc.subcore_barrier()

    def outer(j, carry):
        base = w * NCHUNK + j * IDX_BLK
        pltpu.sync_copy(idxg.at[pl.ds(base, IDX_BLK)], igv)
        pltpu.sync_copy(idxs.at[pl.ds(base, IDX_BLK)], isv)
        pltpu.async_copy(table.at[igv.at[0]], rows0, sem0)

        def body(k, carry2):
            i0 = 2 * k
            pltpu.async_copy(table.at[igv.at[i0 + 1]], rows1, sem1)
            pltpu.make_async_copy(table.at[igv.at[i0]], rows0, sem0).wait()
            pltpu.sync_copy(rows0, acc.at[isv.at[i0]], add=True)

            @pl.when(k < IDX_BLK // 2 - 1)
            def _():
                pltpu.async_copy(table.at[igv.at[i0 + 2]], rows0, sem0)

            pltpu.make_async_copy(table.at[igv.at[i0 + 1]], rows1, sem1).wait()
            pltpu.sync_copy(rows1, acc.at[isv.at[i0 + 1]], add=True)
            return carry2

        lax.fori_loop(0, IDX_BLK // 2, body, 0)
        return carry

    lax.fori_loop(0, NREFILL, outer, 0)
    plsc.subcore_barrier()

    @pl.when(c == 0)
    def _():
        pltpu.sync_copy(acc.at[sl], out.at[0, sl])

    @pl.when(c == 1)
    def _():
        pltpu.sync_copy(acc.at[sl], out.at[1, sl])


def _sc_pass(table, idxg, idxs, zrows):
    return pl.kernel(
        _sc_pass_body,
        out_type=jax.ShapeDtypeStruct((NCORE, N_PAD, H), jnp.float32),
        mesh=plsc.VectorSubcoreMesh(**_SC_MESH),
        scratch_types=[
            pltpu.VMEM_SHARED((N_PAD, H), jnp.float32),
            pltpu.VMEM((IDX_BLK, CHUNK), jnp.int32),
            pltpu.VMEM((IDX_BLK, CHUNK), jnp.int32),
            pltpu.VMEM((CHUNK, H), jnp.float32),
            pltpu.VMEM((CHUNK, H), jnp.float32),
            pltpu.SemaphoreType.DMA,
            pltpu.SemaphoreType.DMA,
        ],
        compiler_params=pltpu.CompilerParams(use_tc_tiling_on_sc=False),
    )(table, idxg, idxs, zrows)


def _sc_counts_body(idxg, idxs, zrows, ones, outd, outb, accd, accb,
                    igv, isv, onev):
    c = lax.axis_index("c")
    s = lax.axis_index("s")
    w = c * NSUB + s
    sl = pl.ds(s * ROWS_PER_TILE, ROWS_PER_TILE)
    pltpu.sync_copy(zrows, accd.at[sl])
    pltpu.sync_copy(zrows, accb.at[sl])
    pltpu.sync_copy(ones, onev)
    plsc.subcore_barrier()

    def outer(j, carry):
        base = w * NCHUNK + j * IDX_BLK
        pltpu.sync_copy(idxg.at[pl.ds(base, IDX_BLK)], igv)
        pltpu.sync_copy(idxs.at[pl.ds(base, IDX_BLK)], isv)

        def body(i, carry2):
            pltpu.sync_copy(onev, accd.at[igv.at[i]], add=True)
            pltpu.sync_copy(onev, accb.at[isv.at[i]], add=True)
            return carry2

        lax.fori_loop(0, IDX_BLK, body, 0)
        return carry

    lax.fori_loop(0, NREFILL, outer, 0)
    plsc.subcore_barrier()

    @pl.when(c == 0)
    def _():
        pltpu.sync_copy(accd.at[sl], outd.at[0, sl])
        pltpu.sync_copy(accb.at[sl], outb.at[0, sl])

    @pl.when(c == 1)
    def _():
        pltpu.sync_copy(accd.at[sl], outd.at[1, sl])
        pltpu.sync_copy(accb.at[sl], outb.at[1, sl])


def _sc_counts(idxg, idxs):
    zrows = jnp.zeros((ROWS_PER_TILE, CW), jnp.float32)
    ones = jnp.ones((CHUNK, CW), jnp.float32)
    return pl.kernel(
        _sc_counts_body,
        out_type=(jax.ShapeDtypeStruct((NCORE, N_PAD, CW), jnp.float32),
                  jax.ShapeDtypeStruct((NCORE, N_PAD, CW), jnp.float32)),
        mesh=plsc.VectorSubcoreMesh(**_SC_MESH),
        scratch_types=[
            pltpu.VMEM_SHARED((N_PAD, CW), jnp.float32),
            pltpu.VMEM_SHARED((N_PAD, CW), jnp.float32),
            pltpu.VMEM((IDX_BLK, CHUNK), jnp.int32),
            pltpu.VMEM((IDX_BLK, CHUNK), jnp.int32),
            pltpu.VMEM((CHUNK, CW), jnp.float32),
        ],
        compiler_params=pltpu.CompilerParams(use_tc_tiling_on_sc=False),
    )(idxg, idxs, zrows, ones)


GBLK = 10000
GGRID = N_NODES // GBLK


def _gru_body(x_ref, wxr_ref, wxz_ref, wxn_ref, whr_ref, whz_ref, whn_ref,
              br_ref, bz_ref, bni_ref, bnh_ref, w1_ref, out_ref):
    x = x_ref[...]
    whr = whr_ref[...]
    whz = whz_ref[...]
    whn = whn_ref[...]
    br = br_ref[...]
    bz = bz_ref[...]
    bni = bni_ref[...]
    bnh = bnh_ref[...]
    h = jnp.zeros((GBLK, H), jnp.float32)
    for t in range(SEQ):
        gr = (jnp.dot(x, wxr_ref[t], preferred_element_type=jnp.float32)
              + jnp.dot(h, whr, preferred_element_type=jnp.float32) + br)
        gz = (jnp.dot(x, wxz_ref[t], preferred_element_type=jnp.float32)
              + jnp.dot(h, whz, preferred_element_type=jnp.float32) + bz)
        r = jax.nn.sigmoid(gr)
        z = jax.nn.sigmoid(gz)
        hn = jnp.dot(h, whn, preferred_element_type=jnp.float32) + bnh
        gn = jnp.dot(x, wxn_ref[t], preferred_element_type=jnp.float32) + bni
        n = jnp.tanh(gn + r * hn)
        h = (1.0 - z) * n + z * h
    o = jnp.where(h > 0, h, 0.01 * h)
    out_ref[...] = jnp.dot(o, w1_ref[...], preferred_element_type=jnp.float32)


def _tc_gru(x2, wx, wh, bg, w1m_t):
    full = lambda shape: pl.BlockSpec(shape, lambda i: tuple(0 for _ in shape))
    return pl.pallas_call(
        _gru_body,
        grid=(GGRID,),
        in_specs=[
            pl.BlockSpec((GBLK, SEQ * FIN), lambda i: (i, 0)),
            full((SEQ, SEQ * FIN, H)),
            full((SEQ, SEQ * FIN, H)),
            full((SEQ, SEQ * FIN, H)),
            full((H, H)),
            full((H, H)),
            full((H, H)),
            full((1, H)),
            full((1, H)),
            full((1, H)),
            full((1, H)),
            full((H, H)),
        ],
        out_specs=pl.BlockSpec((GBLK, H), lambda i: (i, 0)),
        out_shape=jax.ShapeDtypeStruct((N_NODES, H), jnp.float32),
    )(x2, wx[0], wx[1], wx[2], wh[0], wh[1], wh[2],
      bg[0], bg[1], bg[2], bg[3], w1m_t)


def _inv_body(cd_ref, cb_ref, dinv_ref, binv_ref):
    cd = cd_ref[0, :, :1] + cd_ref[1, :, :1]
    cb = cb_ref[0, :, :1] + cb_ref[1, :, :1]
    dinv_ref[...] = jnp.where(cd > 0, 1.0 / cd, 0.0)
    binv_ref[...] = jnp.where(cb > 0, 1.0 / cb, 0.0)


def _tc_inv(cd, cb):
    return pl.pallas_call(
        _inv_body,
        grid=(GRID,),
        in_specs=[
            pl.BlockSpec((NCORE, BLK, CW), lambda i: (0, i, 0)),
            pl.BlockSpec((NCORE, BLK, CW), lambda i: (0, i, 0)),
        ],
        out_specs=[
            pl.BlockSpec((BLK, 1), lambda i: (i, 0)),
            pl.BlockSpec((BLK, 1), lambda i: (i, 0)),
        ],
        out_shape=[jax.ShapeDtypeStruct((N_NODES, 1), jnp.float32),
                   jax.ShapeDtypeStruct((N_NODES, 1), jnp.float32)],
    )(cd, cb)


def _edge_flush_body(p_ref, binv_ref, out_ref):
    out_ref[...] = (p_ref[0] + p_ref[1]) * binv_ref[...]


def _tc_edge_flush(p, binv):
    return pl.pallas_call(
        _edge_flush_body,
        grid=(GRID,),
        in_specs=[
            pl.BlockSpec((NCORE, BLK, H), lambda i: (0, i, 0)),
            pl.BlockSpec((BLK, 1), lambda i: (i, 0)),
        ],
        out_specs=pl.BlockSpec((BLK, H), lambda i: (i, 0)),
        out_shape=jax.ShapeDtypeStruct((N_NODES, H), jnp.float32),
    )(p, binv)


def _node_flush_body(p_ref, dinv_ref, b_ref, w_ref, out_ref):
    y = (p_ref[0] + p_ref[1]) * dinv_ref[...] + b_ref[...]
    y = jnp.where(y > 0, y, 0.2 * y)
    out_ref[...] = jnp.dot(y, w_ref[...], preferred_element_type=jnp.float32)


def _tc_node_flush(p, dinv, b2d, w_t):
    return pl.pallas_call(
        _node_flush_body,
        grid=(GRID,),
        in_specs=[
            pl.BlockSpec((NCORE, BLK, H), lambda i: (0, i, 0)),
            pl.BlockSpec((BLK, 1), lambda i: (i, 0)),
            pl.BlockSpec((1, H), lambda i: (0, 0)),
            pl.BlockSpec((H, H), lambda i: (0, 0)),
        ],
        out_specs=pl.BlockSpec((BLK, H), lambda i: (i, 0)),
        out_shape=jax.ShapeDtypeStruct((N_NODES, H), jnp.float32),
    )(p, dinv, b2d, w_t)


def _final_body(p_ref, dinv_ref, b_ref, wl_ref, bl_ref, out_ref):
    y = (p_ref[0] + p_ref[1]) * dinv_ref[...] + b_ref[...]
    y = jnp.where(y > 0, y, 0.2 * y)
    o = jnp.dot(y, wl_ref[...], preferred_element_type=jnp.float32) + bl_ref[...]
    out_ref[...] = jnp.where(o > 0, o, 0.01 * o)


def _tc_final(p, dinv, b2d, wl_t, bl2d):
    n_out = wl_t.shape[1]
    return pl.pallas_call(
        _final_body,
        grid=(GRID,),
        in_specs=[
            pl.BlockSpec((NCORE, BLK, H), lambda i: (0, i, 0)),
            pl.BlockSpec((BLK, 1), lambda i: (i, 0)),
            pl.BlockSpec((1, H), lambda i: (0, 0)),
            pl.BlockSpec((H, n_out), lambda i: (0, 0)),
            pl.BlockSpec((1, n_out), lambda i: (0, 0)),
        ],
        out_specs=pl.BlockSpec((BLK, n_out), lambda i: (i, 0)),
        out_shape=jax.ShapeDtypeStruct((N_NODES, n_out), jnp.float32),
    )(p, dinv, b2d, wl_t, bl2d)


def kernel(price_input, e, concept, Wih, Whh, bih, bhh, W1, b1, W2, b2, Wl, bl):
    x2 = price_input.reshape(N_NODES, SEQ * FIN)
    w1m_t = W1.reshape(4, H, H).mean(axis=0).T
    wihT = Wih.T
    whhT = Whh.T
    wx = []
    for g in range(3):
        wg = jnp.zeros((SEQ, SEQ * FIN, H), jnp.float32)
        for t in range(SEQ):
            wg = wg.at[t, t * FIN:(t + 1) * FIN, :].set(wihT[:, g * H:(g + 1) * H])
        wx.append(wg)
    wh = [whhT[:, g * H:(g + 1) * H] for g in range(3)]
    bg = [(bih[:H] + bhh[:H]).reshape(1, H),
          (bih[H:2 * H] + bhh[H:2 * H]).reshape(1, H),
          bih[2 * H:].reshape(1, H),
          bhh[2 * H:].reshape(1, H)]
    t1 = _tc_gru(x2, wx, wh, bg, w1m_t)
    ig = e[0].reshape(IDX_ROWS, CHUNK)
    ie = e[1].reshape(IDX_ROWS, CHUNK)
    cd, cb = _sc_counts(ig, ie)
    dinv, binv = _tc_inv(cd, cb)
    z = jnp.zeros((ROWS_PER_TILE, H), jnp.float32)
    p = _sc_pass(t1, ig, ie, z)
    t2 = _tc_edge_flush(p, binv)
    p = _sc_pass(t2, ie, ig, z)
    t3 = _tc_node_flush(p, dinv, b1.reshape(1, -1), W2.T)
    p = _sc_pass(t3, ig, ie, z)
    t4 = _tc_edge_flush(p, binv)
    p = _sc_pass(t4, ie, ig, z)
    return _tc_final(p, dinv, b2.reshape(1, -1), Wl.T, bl.reshape(1, -1))
